# Initial kernel scaffold; baseline (speedup 1.0000x reference)
import jax
import jax.numpy as jnp
from jax.experimental import pallas as pl


def kernel(input, char_emb, conv_w, conv_b, w_proj, b_proj, w_gate, b_gate):
    raise NotImplementedError("write your pallas kernel here")


# speedup vs baseline: 7.6665x; 7.6665x over previous
import jax
import jax.numpy as jnp
from jax.experimental import pallas as pl

S, B, W = 20, 1024, 21
V, CE, F = 96, 50, 128
K = 5
T = W - K + 1
N = S * B
NB = 256
CEP = 64


def _fused_body(idx_ref, emb_ref, wk_ref, cb_ref, wp_ref, bp_ref, wg_ref,
                bg_ref, out_ref):
    idx = idx_ref[...]
    iot = jax.lax.broadcasted_iota(jnp.int32, (W, NB, 128), 2)
    oh = (idx[:, :, None] == iot).astype(jnp.float32)
    oh2 = oh.reshape(W * NB, 128)
    xs = jnp.dot(oh2, emb_ref[...], preferred_element_type=jnp.float32)
    acc = jnp.zeros((T * NB, F), jnp.float32)
    for k in range(K):
        acc = acc + jnp.dot(xs[k * NB:(k + T) * NB, :],
                            wk_ref[k * CEP:(k + 1) * CEP, :],
                            preferred_element_type=jnp.float32)
    conv = jnp.maximum(acc + cb_ref[...], 0.0)
    m = jnp.max(conv.reshape(T, NB, F), axis=0)
    hp = jnp.maximum(
        jnp.dot(m, wp_ref[...], preferred_element_type=jnp.float32)
        + bp_ref[...], 0.0)
    hg = jax.nn.sigmoid(
        jnp.dot(m, wg_ref[...], preferred_element_type=jnp.float32)
        + bg_ref[...])
    out_ref[...] = hg * hp + (1.0 - hg) * m


def kernel(input, char_emb, conv_w, conv_b, w_proj, b_proj, w_gate, b_gate):
    idxp = input.reshape(N, W).T
    emb_pad = jnp.zeros((128, CEP), jnp.float32).at[:V, :CE].set(char_emb)
    wk = jnp.transpose(conv_w, (2, 1, 0))
    wk_all = (jnp.zeros((K, CEP, F), jnp.float32).at[:, :CE, :].set(wk)
              .reshape(K * CEP, F))
    cb2 = conv_b.reshape(1, F)
    bp2 = b_proj.reshape(1, F)
    bg2 = b_gate.reshape(1, F)

    out = pl.pallas_call(
        _fused_body,
        grid=(N // NB,),
        in_specs=[
            pl.BlockSpec((W, NB), lambda i: (0, i)),
            pl.BlockSpec((128, CEP), lambda i: (0, 0)),
            pl.BlockSpec((K * CEP, F), lambda i: (0, 0)),
            pl.BlockSpec((1, F), lambda i: (0, 0)),
            pl.BlockSpec((F, F), lambda i: (0, 0)),
            pl.BlockSpec((1, F), lambda i: (0, 0)),
            pl.BlockSpec((F, F), lambda i: (0, 0)),
            pl.BlockSpec((1, F), lambda i: (0, 0)),
        ],
        out_specs=pl.BlockSpec((NB, F), lambda i: (i, 0)),
        out_shape=jax.ShapeDtypeStruct((N, F), jnp.float32),
    )(idxp, emb_pad, wk_all, cb2, w_proj.T, bp2, w_gate.T, bg2)
    return out.reshape(S, B, F)
